# baseline (speedup 1.0000x reference)
import jax
import jax.numpy as jnp
from jax.experimental import pallas as pl
from jax.experimental.pallas import tpu as pltpu

_NE = 8
_K = 2
_AUX_COEF = 0.01


def _router_body(x_ref, w_ref, rw_ref, se_ref, acc_ref, aux_ref, *, ntok):
    step = pl.program_id(0)
    nsteps = pl.num_programs(0)

    x = x_ref[...]
    w = w_ref[...]
    logits = jax.lax.dot_general(
        x, w, (((1,), (1,)), ((), ())), preferred_element_type=jnp.float32
    )

    m = jnp.max(logits, axis=-1, keepdims=True)
    e = jnp.exp(logits - m)
    s = jnp.sum(e, axis=-1, keepdims=True)
    probs = e / s

    lane = jax.lax.broadcasted_iota(jnp.int32, probs.shape, 1)
    m1 = jnp.max(probs, axis=-1, keepdims=True)
    i1 = jnp.min(jnp.where(probs == m1, lane, _NE), axis=-1, keepdims=True)
    masked = jnp.where(lane == i1, -1.0, probs)
    m2 = jnp.max(masked, axis=-1, keepdims=True)
    i2 = jnp.min(jnp.where(masked == m2, lane, _NE), axis=-1, keepdims=True)

    tot = m1 + m2
    rw_ref[...] = jnp.concatenate([m1 / tot, m2 / tot], axis=1)
    se_ref[...] = jnp.concatenate([i1, i2], axis=1)

    psum = jnp.sum(probs, axis=0, keepdims=True)
    csum = jnp.sum(
        (lane == i1).astype(jnp.float32) + (lane == i2).astype(jnp.float32),
        axis=0, keepdims=True,
    )
    part = jnp.concatenate([psum, csum], axis=0)

    @pl.when(step == 0)
    def _init():
        acc_ref[...] = jnp.zeros_like(acc_ref)

    acc_ref[...] += part

    @pl.when(step == nsteps - 1)
    def _fin():
        a = acc_ref[...]
        router_frac = a[0:1, :] * (1.0 / ntok)
        expert_frac = a[1:2, :] * (1.0 / (ntok * _K))
        aux_ref[...] = (_NE * _AUX_COEF) * jnp.sum(
            router_frac * expert_frac, axis=-1, keepdims=True
        )


def kernel(hidden_states, W_gate):
    b, sq, d = hidden_states.shape
    ntok = b * sq
    x = hidden_states.reshape(ntok, d)

    T = 2048
    grid = ntok // T

    import functools
    body = functools.partial(_router_body, ntok=ntok)

    rw, se, _, aux = pl.pallas_call(
        body,
        grid=(grid,),
        in_specs=[
            pl.BlockSpec((T, d), lambda i: (i, 0)),
            pl.BlockSpec((_NE, d), lambda i: (0, 0)),
        ],
        out_specs=[
            pl.BlockSpec((T, _K), lambda i: (i, 0)),
            pl.BlockSpec((T, _K), lambda i: (i, 0)),
            pl.BlockSpec((2, _NE), lambda i: (0, 0)),
            pl.BlockSpec((1, 1), lambda i: (0, 0)),
        ],
        out_shape=[
            jax.ShapeDtypeStruct((ntok, _K), jnp.float32),
            jax.ShapeDtypeStruct((ntok, _K), jnp.int32),
            jax.ShapeDtypeStruct((2, _NE), jnp.float32),
            jax.ShapeDtypeStruct((1, 1), jnp.float32),
        ],
        compiler_params=pltpu.CompilerParams(
            dimension_semantics=("arbitrary",),
        ),
    )(x, W_gate)

    return rw.reshape(b, sq, _K), se.reshape(b, sq, _K), aux[0, 0]


# speedup vs baseline: 1.4152x; 1.4152x over previous
import functools

import jax
import jax.numpy as jnp
from jax.experimental import pallas as pl
from jax.experimental.pallas import tpu as pltpu

_NE = 8
_K = 2
_AUX_COEF = 0.01


def _router_body(x_ref, w_ref, rw_ref, se_ref, acc_ref, aux_ref, *, ntok):
    step = pl.program_id(0)
    nsteps = pl.num_programs(0)

    x = x_ref[...]
    w = w_ref[...]
    logits = jax.lax.dot_general(
        w, x, (((1,), (1,)), ((), ())), preferred_element_type=jnp.float32
    )

    m = jnp.max(logits, axis=0, keepdims=True)
    e = jnp.exp(logits - m)
    s = jnp.sum(e, axis=0, keepdims=True)
    probs = e / s

    sub = jax.lax.broadcasted_iota(jnp.int32, probs.shape, 0)
    m1 = jnp.max(probs, axis=0, keepdims=True)
    i1 = jnp.min(jnp.where(probs == m1, sub, _NE), axis=0, keepdims=True)
    masked = jnp.where(sub == i1, -1.0, probs)
    m2 = jnp.max(masked, axis=0, keepdims=True)
    i2 = jnp.min(jnp.where(masked == m2, sub, _NE), axis=0, keepdims=True)

    inv_tot = 1.0 / (m1 + m2)
    rw_ref[...] = jnp.concatenate([m1 * inv_tot, m2 * inv_tot], axis=0)
    se_ref[...] = jnp.concatenate([i1, i2], axis=0)

    psum = jnp.sum(probs, axis=1, keepdims=True)
    csum = jnp.sum(
        (sub == i1).astype(jnp.float32) + (sub == i2).astype(jnp.float32),
        axis=1, keepdims=True,
    )
    part = jnp.concatenate([psum, csum], axis=1)

    @pl.when(step == 0)
    def _init():
        acc_ref[...] = jnp.zeros_like(acc_ref)

    acc_ref[...] += part

    @pl.when(step == nsteps - 1)
    def _fin():
        a = acc_ref[...]
        prod = a[:, 0:1] * a[:, 1:2]
        scale = (_NE * _AUX_COEF) / (float(ntok) * float(ntok) * _K)
        aux_ref[...] = scale * jnp.sum(prod, axis=0, keepdims=True)


def kernel(hidden_states, W_gate):
    b, sq, d = hidden_states.shape
    ntok = b * sq
    x = hidden_states.reshape(ntok, d)

    T = 2048
    grid = ntok // T

    body = functools.partial(_router_body, ntok=ntok)

    rw, se, _, aux = pl.pallas_call(
        body,
        grid=(grid,),
        in_specs=[
            pl.BlockSpec((T, d), lambda i: (i, 0)),
            pl.BlockSpec((_NE, d), lambda i: (0, 0)),
        ],
        out_specs=[
            pl.BlockSpec((_K, T), lambda i: (0, i)),
            pl.BlockSpec((_K, T), lambda i: (0, i)),
            pl.BlockSpec((_NE, 2), lambda i: (0, 0)),
            pl.BlockSpec((1, 1), lambda i: (0, 0)),
        ],
        out_shape=[
            jax.ShapeDtypeStruct((_K, ntok), jnp.float32),
            jax.ShapeDtypeStruct((_K, ntok), jnp.int32),
            jax.ShapeDtypeStruct((_NE, 2), jnp.float32),
            jax.ShapeDtypeStruct((1, 1), jnp.float32),
        ],
        compiler_params=pltpu.CompilerParams(
            dimension_semantics=("arbitrary",),
        ),
    )(x, W_gate)

    rw = rw.T.reshape(b, sq, _K)
    se = se.T.reshape(b, sq, _K)
    return rw, se, aux[0, 0]
